# baseline (speedup 1.0000x reference)
import functools

import jax
import jax.numpy as jnp
from jax import lax
from jax.experimental import pallas as pl
from jax.experimental.pallas import tpu as pltpu
from jax.experimental.pallas import tpu_sc as plsc

N = 100000
D = 128
B = 256
H = 4

RB = 4096
NB = -(-N // RB)
NPAD = NB * RB
RB2 = 8192
NB2 = -(-N // RB2)
NPAD2 = NB2 * RB2
SLEN = 256
T = 256
GT = 288
NSEG = B // 32
NC = 2
NS = 16


def _gate_body(x_ref, wt_ref, bias_ref, ids_ref, svals_ref,
               gate_ref, starts_ref, xb_ref):
    i = pl.program_id(0)
    x = x_ref[...]
    xb_ref[...] = x.astype(jnp.bfloat16)
    logitst = lax.dot_general(
        wt_ref[...], x, (((1,), (1,)), ((), ())),
        preferred_element_type=jnp.float32)
    sigt = jax.nn.sigmoid(logitst + bias_ref[...])
    gate = jnp.sum(sigt, axis=0) * (1.0 / H) - 0.5
    gate_ref[0, 0, :] = gate

    ids = ids_ref[0, 0, :]
    sv = svals_ref[...]
    one = jnp.ones((), jnp.bfloat16)
    hi = jnp.floor(ids * jnp.bfloat16(0.0625))
    lo = ids - jnp.bfloat16(16) * hi
    cmph = jnp.where(hi[None, :] < sv, one, 0 * one)
    eqh = jnp.where(hi[None, :] == sv, one, 0 * one)
    cmpl = jnp.where(lo[None, :] < sv, one, 0 * one)
    chb = jnp.dot(cmph, jnp.ones((RB, 16), jnp.bfloat16),
                  preferred_element_type=jnp.float32)
    m = lax.dot_general(eqh, cmpl, (((1,), (1,)), ((), ())),
                        preferred_element_type=jnp.float32)
    cnt = chb + m

    @pl.when(i == 0)
    def _():
        starts_ref[0] = cnt

    @pl.when(i > 0)
    def _():
        starts_ref[0] += cnt


def _tc_stage(feats, wt, bias, ids3, svals):
    _out = pl.pallas_call(
        _gate_body,
        grid=(NB,),
        in_specs=[
            pl.BlockSpec((RB, D), lambda i: (i, 0)),
            pl.BlockSpec((8, D), lambda i: (0, 0)),
            pl.BlockSpec((8, RB), lambda i: (0, 0)),
            pl.BlockSpec((1, 1, RB), lambda i: (i, 0, 0)),
            pl.BlockSpec((16, 1), lambda i: (0, 0)),
        ],
        out_specs=[
            pl.BlockSpec((1, 1, RB), lambda i: (i, 0, 0)),
            pl.BlockSpec((1, 16, 16), lambda i: (0, 0, 0)),
            pl.BlockSpec((RB, D), lambda i: (i, 0)),
        ],
        out_shape=[
            jax.ShapeDtypeStruct((NB, 1, RB), jnp.float32),
            jax.ShapeDtypeStruct((1, 16, 16), jnp.float32),
            jax.ShapeDtypeStruct((NPAD, D), jnp.bfloat16),
        ],
        compiler_params=pltpu.CompilerParams(
            dimension_semantics=("arbitrary",)),
    )(feats, wt, bias, ids3, svals)
    gate3, starts3, xb = _out
    return gate3.reshape(NPAD), starts3.reshape(SLEN), xb


def _sc_body(xb_hbm, gate_hbm, starts_hbm, out_hbm,
             startbuf, rowbuf, gatebuf, rsem, gsem, outbuf):
    wid = lax.axis_index("s") * NC + lax.axis_index("c")
    seg0 = wid * NSEG
    pltpu.sync_copy(starts_hbm.at[pl.ds(seg0, 16)], startbuf)
    sv = startbuf[...].astype(jnp.int32)

    def issue(r0, t, slot):
        a = r0 + t * T
        ga = (a // 8) * 8
        pltpu.make_async_copy(
            xb_hbm.at[pl.ds(a, T)], rowbuf.at[slot], rsem).start()
        pltpu.make_async_copy(
            gate_hbm.at[pl.ds(ga, GT)], gatebuf.at[slot, pl.ds(0, GT)],
            gsem).start()

    def wait_tile():
        pltpu.make_async_copy(
            xb_hbm.at[pl.ds(0, T)], rowbuf.at[0], rsem).wait()
        pltpu.make_async_copy(
            gate_hbm.at[pl.ds(0, GT)], gatebuf.at[0, pl.ds(0, GT)],
            gsem).wait()

    for i in range(NSEG):
        r0 = sv[i]
        r1 = sv[i + 1]
        cnt = r1 - r0
        nt = (cnt + (T - 1)) // T

        @pl.when(nt > 0)
        def _():
            issue(r0, 0, 0)

        def tile_body(t, accs):
            slot = lax.rem(t, 2)

            @pl.when(t + 1 < nt)
            def _():
                issue(r0, t + 1, 1 - slot)

            wait_tile()
            a = r0 + t * T
            off = a - (a // 8) * 8
            nrows = jnp.minimum(cnt - t * T, T)

            def row_body(r, accs2):
                se, so, me, mo = accs2
                g = gatebuf[slot, pl.ds(r + off, 16)][0]
                nse, nso, nme, nmo = [], [], [], []
                for k in range(4):
                    w = rowbuf[slot, r, pl.ds(k * 32, 32)]
                    vi = plsc.bitcast(w, jnp.int32)
                    fe = plsc.bitcast(
                        lax.shift_left(vi, 16), jnp.float32)
                    fo = plsc.bitcast(vi, jnp.float32)
                    nse.append(se[k] + g * fe)
                    nso.append(so[k] + g * fo)
                    nme.append(jnp.maximum(me[k], fe))
                    nmo.append(jnp.maximum(mo[k], fo))
                return (tuple(nse), tuple(nso), tuple(nme), tuple(nmo))

            return lax.fori_loop(0, nrows, row_body, accs)

        zero = jnp.zeros((16,), jnp.float32)
        ninf = jnp.full((16,), -jnp.inf, jnp.float32)
        se, so, me, mo = lax.fori_loop(
            0, nt, tile_body,
            ((zero,) * 4, (zero,) * 4, (ninf,) * 4, (ninf,) * 4))
        iota = lax.iota(jnp.int32, 16)
        row_i = jnp.full((16,), i, jnp.int32)
        for k in range(4):
            idx_e = 32 * k + 2 * iota
            plsc.store_scatter(outbuf, [row_i, idx_e], se[k])
            plsc.store_scatter(outbuf, [row_i, idx_e + 1], so[k])
            plsc.store_scatter(outbuf, [row_i, idx_e + D], me[k])
            plsc.store_scatter(outbuf, [row_i, idx_e + D + 1], mo[k])

    pltpu.sync_copy(outbuf, out_hbm.at[pl.ds(seg0, NSEG)])


_sc_stage = functools.partial(
    pl.kernel,
    out_type=jax.ShapeDtypeStruct((B, 2 * D), jnp.float32),
    mesh=plsc.VectorSubcoreMesh(core_axis_name="c", subcore_axis_name="s"),
    compiler_params=pltpu.CompilerParams(
        use_tc_tiling_on_sc=False, needs_layout_passes=False),
    scratch_types=[
        pltpu.VMEM((16,), jnp.float32),
        pltpu.VMEM((2, T, D), jnp.bfloat16),
        pltpu.VMEM((2, GT + 16), jnp.float32),
        pltpu.SemaphoreType.DMA,
        pltpu.SemaphoreType.DMA,
        pltpu.VMEM((NSEG, 2 * D), jnp.float32),
    ],
)(_sc_body)


def kernel(feats, segment_ids, W, b):
    ids3 = jnp.concatenate(
        [segment_ids.astype(jnp.bfloat16),
         jnp.full((NPAD - N,), 512.0, jnp.bfloat16)]
    ).reshape(NB, 1, RB)
    svals = jnp.arange(16, dtype=jnp.float32).astype(
        jnp.bfloat16).reshape(16, 1)
    wt = jnp.pad(W, ((0, 8 - H), (0, 0)))
    bias = jnp.broadcast_to(jnp.pad(b, (0, 8 - H))[:, None], (8, RB))
    gate, starts, xb = _tc_stage(feats, wt, bias, ids3, svals)
    starts_full = jnp.concatenate(
        [starts, jnp.full((8,), float(N), jnp.float32)])
    return _sc_stage(xb, gate, starts_full)


# speedup vs baseline: 2.0937x; 2.0937x over previous
import functools

import jax
import jax.numpy as jnp
from jax import lax
from jax.experimental import pallas as pl
from jax.experimental.pallas import tpu as pltpu
from jax.experimental.pallas import tpu_sc as plsc

N = 100000
D = 128
B = 256
H = 4

RB = 4096
NB = -(-N // RB)
NPAD = NB * RB
RB2 = 8192
NB2 = -(-N // RB2)
NPAD2 = NB2 * RB2
SLEN = 256
T = 256
GT = 288
NSEG = B // 32
NC = 2
NS = 16


def _gate_body(x_ref, wt_ref, bias_ref, ids_ref, svals_ref,
               gate_ref, starts_ref):
    i = pl.program_id(0)
    x = x_ref[...]
    logitst = lax.dot_general(
        wt_ref[...], x, (((1,), (1,)), ((), ())),
        preferred_element_type=jnp.float32)
    sigt = jax.nn.sigmoid(logitst + bias_ref[...])
    gate = jnp.sum(sigt, axis=0) * (1.0 / H) - 0.5
    gate_ref[0, 0, :] = gate

    ids = ids_ref[0, 0, :]
    sv = svals_ref[...]
    one = jnp.ones((), jnp.bfloat16)
    hi = jnp.floor(ids * jnp.bfloat16(0.0625))
    lo = ids - jnp.bfloat16(16) * hi
    cmph = jnp.where(hi[None, :] < sv, one, 0 * one)
    eqh = jnp.where(hi[None, :] == sv, one, 0 * one)
    cmpl = jnp.where(lo[None, :] < sv, one, 0 * one)
    chb = jnp.dot(cmph, jnp.ones((RB, 16), jnp.bfloat16),
                  preferred_element_type=jnp.float32)
    m = lax.dot_general(eqh, cmpl, (((1,), (1,)), ((), ())),
                        preferred_element_type=jnp.float32)
    cnt = chb + m

    @pl.when(i == 0)
    def _():
        starts_ref[0] = cnt

    @pl.when(i > 0)
    def _():
        starts_ref[0] += cnt


def _tc_stage(feats, wt, bias, ids3, svals):
    _out = pl.pallas_call(
        _gate_body,
        grid=(NB,),
        in_specs=[
            pl.BlockSpec((RB, D), lambda i: (i, 0)),
            pl.BlockSpec((8, D), lambda i: (0, 0)),
            pl.BlockSpec((8, RB), lambda i: (0, 0)),
            pl.BlockSpec((1, 1, RB), lambda i: (i, 0, 0)),
            pl.BlockSpec((16, 1), lambda i: (0, 0)),
        ],
        out_specs=[
            pl.BlockSpec((1, 1, RB), lambda i: (i, 0, 0)),
            pl.BlockSpec((1, 16, 16), lambda i: (0, 0, 0)),
        ],
        out_shape=[
            jax.ShapeDtypeStruct((NB, 1, RB), jnp.float32),
            jax.ShapeDtypeStruct((1, 16, 16), jnp.float32),
        ],
        compiler_params=pltpu.CompilerParams(
            dimension_semantics=("arbitrary",)),
    )(feats, wt, bias, ids3, svals)
    gate3, starts3 = _out
    return gate3.reshape(NPAD), starts3.reshape(SLEN)


def _sc_body(feats_hbm, gate_hbm, starts_hbm, out_hbm,
             startbuf, rowbuf, gatebuf, rsem, gsem, outbuf):
    wid = lax.axis_index("s") * NC + lax.axis_index("c")
    seg0 = wid * NSEG
    pltpu.sync_copy(starts_hbm.at[pl.ds(seg0, 16)], startbuf)
    sv = startbuf[...].astype(jnp.int32)

    def issue(r0, t, slot):
        a = r0 + t * T
        fa = jnp.minimum(a, N - T)
        ga = (a // 8) * 8
        pltpu.make_async_copy(
            feats_hbm.at[pl.ds(fa, T)], rowbuf.at[slot], rsem).start()
        pltpu.make_async_copy(
            gate_hbm.at[pl.ds(ga, GT)], gatebuf.at[slot, pl.ds(0, GT)],
            gsem).start()

    def wait_tile():
        pltpu.make_async_copy(
            feats_hbm.at[pl.ds(0, T)], rowbuf.at[0], rsem).wait()
        pltpu.make_async_copy(
            gate_hbm.at[pl.ds(0, GT)], gatebuf.at[0, pl.ds(0, GT)],
            gsem).wait()

    for i in range(NSEG):
        r0 = sv[i]
        r1 = sv[i + 1]
        cnt = r1 - r0
        nt = (cnt + (T - 1)) // T

        @pl.when(nt > 0)
        def _():
            issue(r0, 0, 0)

        def tile_body(t, accs):
            slot = lax.rem(t, 2)

            @pl.when(t + 1 < nt)
            def _():
                issue(r0, t + 1, 1 - slot)

            wait_tile()
            a = r0 + t * T
            d = a - jnp.minimum(a, N - T)
            off = a - (a // 8) * 8
            nrows = jnp.minimum(cnt - t * T, T)

            def row_body(r, accs2):
                sums, maxs = accs2
                g = gatebuf[slot, pl.ds(r + off, 16)][0]
                ns, nm = [], []
                for k in range(8):
                    v = rowbuf[slot, r + d, pl.ds(k * 16, 16)]
                    ns.append(sums[k] + g * v)
                    nm.append(jnp.maximum(maxs[k], v))
                return (tuple(ns), tuple(nm))

            return lax.fori_loop(0, nrows, row_body, accs)

        zero = jnp.zeros((16,), jnp.float32)
        ninf = jnp.full((16,), -jnp.inf, jnp.float32)
        sums, maxs = lax.fori_loop(
            0, nt, tile_body, ((zero,) * 8, (ninf,) * 8))
        for k in range(8):
            outbuf[i, pl.ds(k * 16, 16)] = sums[k]
            outbuf[i, pl.ds(D + k * 16, 16)] = maxs[k]

    pltpu.sync_copy(outbuf, out_hbm.at[pl.ds(seg0, NSEG)])


_sc_stage = functools.partial(
    pl.kernel,
    out_type=jax.ShapeDtypeStruct((B, 2 * D), jnp.float32),
    mesh=plsc.VectorSubcoreMesh(core_axis_name="c", subcore_axis_name="s"),
    compiler_params=pltpu.CompilerParams(use_tc_tiling_on_sc=False),
    scratch_types=[
        pltpu.VMEM((16,), jnp.float32),
        pltpu.VMEM((2, T, D), jnp.float32),
        pltpu.VMEM((2, GT + 16), jnp.float32),
        pltpu.SemaphoreType.DMA,
        pltpu.SemaphoreType.DMA,
        pltpu.VMEM((NSEG, 2 * D), jnp.float32),
    ],
)(_sc_body)


def kernel(feats, segment_ids, W, b):
    ids3 = jnp.concatenate(
        [segment_ids.astype(jnp.bfloat16),
         jnp.full((NPAD - N,), 512.0, jnp.bfloat16)]
    ).reshape(NB, 1, RB)
    svals = jnp.arange(16, dtype=jnp.float32).astype(
        jnp.bfloat16).reshape(16, 1)
    wt = jnp.pad(W, ((0, 8 - H), (0, 0)))
    bias = jnp.broadcast_to(jnp.pad(b, (0, 8 - H))[:, None], (8, RB))
    gate, starts = _tc_stage(feats, wt, bias, ids3, svals)
    starts_full = jnp.concatenate(
        [starts, jnp.full((8,), float(N), jnp.float32)])
    return _sc_stage(feats, gate, starts_full)
